# baseline (speedup 1.0000x reference)
import functools

import jax
import jax.numpy as jnp
from jax import lax
from jax.experimental import pallas as pl
from jax.experimental.pallas import tpu as pltpu
from jax.experimental.pallas import tpu_sc as plsc

DIM = 128
SEQ = 50

NC = 2
NS = 16
NW = NC * NS

K = 128
NBUF = 6


def _body(b_per_w, n_chunks,
          ids_hbm, remap_hbm, emb_hbm, out_hbm,
          ids_v, dict_v, r0, r1, r2, r3, r4, r5,
          sr0, sr1, sr2, sr3, sr4, sr5,
          sg0, sg1, sg2, sg3, sg4, sg5,
          ss0, ss1, ss2, ss3, ss4, ss5):
    rows = (r0, r1, r2, r3, r4, r5)
    sr = (sr0, sr1, sr2, sr3, sr4, sr5)
    sg = (sg0, sg1, sg2, sg3, sg4, sg5)
    ss = (ss0, ss1, ss2, ss3, ss4, ss5)

    wid = lax.axis_index("s") * NC + lax.axis_index("c")
    base = wid * b_per_w
    base_b = wid * K

    pltpu.sync_copy(ids_hbm.at[pl.ds(base, b_per_w)], ids_v)

    def fire_r(j, slot):
        off = pl.multiple_of(j * K, K)
        pltpu.async_copy(remap_hbm.at[ids_v.at[pl.ds(off, K)]],
                         dict_v.at[pl.ds(off, K)], sr[slot])

    def wait_r(slot):
        pltpu.make_async_copy(remap_hbm.at[ids_v.at[pl.ds(0, K)]],
                              dict_v.at[pl.ds(0, K)], sr[slot]).wait()

    def issue_g(j, slot, dyn_guard):
        wait_r(slot)
        off = pl.multiple_of(j * K, K)
        pltpu.async_copy(emb_hbm.at[dict_v.at[pl.ds(off, K)]],
                         rows[slot], sg[slot])
        if dyn_guard:
            @pl.when(j + NBUF < n_chunks)
            def _():
                fire_r(j + NBUF, slot)
        else:
            if isinstance(j, int) and j + NBUF < n_chunks:
                fire_r(j + NBUF, slot)

    def wait_g(slot):
        pltpu.make_async_copy(emb_hbm.at[dict_v.at[pl.ds(0, K)]],
                              rows[slot], sg[slot]).wait()

    def issue_s(j, slot):
        pltpu.async_copy(rows[slot], out_hbm.at[j, pl.ds(base_b, K)],
                         ss[slot])

    def wait_s(slot):
        pltpu.make_async_copy(rows[slot], out_hbm.at[0, pl.ds(base_b, K)],
                              ss[slot]).wait()

    for m in range(NBUF):
        fire_r(m, m)

    def head_step(j):
        slot = j % NBUF
        wait_g(slot)
        issue_s(j, slot)
        if j > 0:
            wait_s((j - 1) % NBUF)
        issue_g(j + NBUF - 1, (j + NBUF - 1) % NBUF, False)

    head = 1
    while (n_chunks - NBUF + 1 - head) % NBUF:
        head += 1
    for m in range(NBUF - 1):
        issue_g(m, m, False)
    for j in range(head):
        head_step(j)

    def group(g, c):
        for b in range(NBUF):
            j = g * NBUF + head + b
            slot = (head + b) % NBUF
            wait_g(slot)
            issue_s(j, slot)
            wait_s((slot - 1) % NBUF)
            issue_g(j + NBUF - 1, (slot - 1) % NBUF, True)
        return c

    lax.fori_loop(0, (n_chunks - NBUF + 1 - head) // NBUF, group, 0)

    for j in range(n_chunks - NBUF + 1, n_chunks):
        slot = j % NBUF
        wait_g(slot)
        issue_s(j, slot)
        wait_s((j - 1) % NBUF)
    wait_s((n_chunks - 1) % NBUF)


@jax.jit
def _lookup(ids_perm, remap, emb):
    n_tok = ids_perm.shape[0]
    batch = n_tok // SEQ
    b_per_w = n_tok // NW
    n_chunks = b_per_w // K
    assert n_tok == b_per_w * NW and b_per_w == n_chunks * K
    assert batch == NW * K and n_chunks == SEQ and n_chunks >= 2 * NBUF
    mesh = plsc.VectorSubcoreMesh(core_axis_name="c", subcore_axis_name="s")
    fn = pl.kernel(
        functools.partial(_body, b_per_w, n_chunks),
        out_type=jax.ShapeDtypeStruct((SEQ, batch, DIM), jnp.float32),
        mesh=mesh,
        scratch_types=(
            [pltpu.VMEM((b_per_w,), jnp.int32),
             pltpu.VMEM((b_per_w,), jnp.int32)]
            + [pltpu.VMEM((K, DIM), jnp.float32)] * NBUF
            + [pltpu.SemaphoreType.DMA] * (3 * NBUF)
        ),
    )
    return fn(ids_perm, remap, emb)


def kernel(input_ids, attention_mask, embedding_dict, input_ids2dict_ids):
    batch, seq = input_ids.shape
    ids_perm = (input_ids.astype(jnp.int32)
                .reshape(NW, K, seq).transpose(0, 2, 1).reshape(-1))
    remap = input_ids2dict_ids.astype(jnp.int32)
    out_t = _lookup(ids_perm, remap, embedding_dict)
    return (out_t.transpose(1, 0, 2), attention_mask)


# speedup vs baseline: 1.0397x; 1.0397x over previous
import functools

import jax
import jax.numpy as jnp
from jax import lax
from jax.experimental import pallas as pl
from jax.experimental.pallas import tpu as pltpu
from jax.experimental.pallas import tpu_sc as plsc

DIM = 128
SEQ = 50

NC = 2
NS = 16
NW = NC * NS

K = 128
NBUF = 6


def _body(b_per_w, n_chunks, n_remap,
          ids_hbm, remap_hbm, emb_hbm, out_hbm,
          ids_v, dict_v, bounce_v, remap_sp, r0, r1, r2, r3, r4, r5,
          sr0, sr1, sr2, sr3, sr4, sr5,
          sg0, sg1, sg2, sg3, sg4, sg5,
          ss0, ss1, ss2, ss3, ss4, ss5):
    rows = (r0, r1, r2, r3, r4, r5)
    sr = (sr0, sr1, sr2, sr3, sr4, sr5)
    sg = (sg0, sg1, sg2, sg3, sg4, sg5)
    ss = (ss0, ss1, ss2, ss3, ss4, ss5)

    wid = lax.axis_index("s") * NC + lax.axis_index("c")
    base = wid * b_per_w
    base_b = wid * K

    sid = lax.axis_index("s")
    seg = n_remap // NS
    soff = pl.multiple_of(sid * seg, seg)
    pltpu.sync_copy(remap_hbm.at[pl.ds(soff, seg)], bounce_v)
    pltpu.sync_copy(bounce_v, remap_sp.at[pl.ds(soff, seg)])

    pltpu.sync_copy(ids_hbm.at[pl.ds(base, b_per_w)], ids_v)
    plsc.subcore_barrier()

    def fire_r(j, slot):
        off = pl.multiple_of(j * K, K)
        pltpu.async_copy(remap_sp.at[ids_v.at[pl.ds(off, K)]],
                         dict_v.at[pl.ds(off, K)], sr[slot])

    def wait_r(slot):
        pltpu.make_async_copy(remap_sp.at[ids_v.at[pl.ds(0, K)]],
                              dict_v.at[pl.ds(0, K)], sr[slot]).wait()

    def issue_g(j, slot, dyn_guard):
        wait_r(slot)
        off = pl.multiple_of(j * K, K)
        pltpu.async_copy(emb_hbm.at[dict_v.at[pl.ds(off, K)]],
                         rows[slot], sg[slot])
        if dyn_guard:
            @pl.when(j + NBUF < n_chunks)
            def _():
                fire_r(j + NBUF, slot)
        else:
            if isinstance(j, int) and j + NBUF < n_chunks:
                fire_r(j + NBUF, slot)

    def wait_g(slot):
        pltpu.make_async_copy(emb_hbm.at[dict_v.at[pl.ds(0, K)]],
                              rows[slot], sg[slot]).wait()

    def issue_s(j, slot):
        pltpu.async_copy(rows[slot], out_hbm.at[j, pl.ds(base_b, K)],
                         ss[slot])

    def wait_s(slot):
        pltpu.make_async_copy(rows[slot], out_hbm.at[0, pl.ds(base_b, K)],
                              ss[slot]).wait()

    for m in range(NBUF):
        fire_r(m, m)

    def head_step(j):
        slot = j % NBUF
        wait_g(slot)
        issue_s(j, slot)
        if j > 0:
            wait_s((j - 1) % NBUF)
        issue_g(j + NBUF - 1, (j + NBUF - 1) % NBUF, False)

    head = 1
    while (n_chunks - NBUF + 1 - head) % NBUF:
        head += 1
    for m in range(NBUF - 1):
        issue_g(m, m, False)
    for j in range(head):
        head_step(j)

    def group(g, c):
        for b in range(NBUF):
            j = g * NBUF + head + b
            slot = (head + b) % NBUF
            wait_g(slot)
            issue_s(j, slot)
            wait_s((slot - 1) % NBUF)
            issue_g(j + NBUF - 1, (slot - 1) % NBUF, True)
        return c

    lax.fori_loop(0, (n_chunks - NBUF + 1 - head) // NBUF, group, 0)

    for j in range(n_chunks - NBUF + 1, n_chunks):
        slot = j % NBUF
        wait_g(slot)
        issue_s(j, slot)
        wait_s((j - 1) % NBUF)
    wait_s((n_chunks - 1) % NBUF)


@jax.jit
def _lookup(ids_perm, remap, emb):
    n_tok = ids_perm.shape[0]
    n_remap = remap.shape[0]
    batch = n_tok // SEQ
    b_per_w = n_tok // NW
    n_chunks = b_per_w // K
    assert n_tok == b_per_w * NW and b_per_w == n_chunks * K
    assert batch == NW * K and n_chunks == SEQ and n_chunks >= 2 * NBUF
    assert n_remap % (8 * NS) == 0
    mesh = plsc.VectorSubcoreMesh(core_axis_name="c", subcore_axis_name="s")
    fn = pl.kernel(
        functools.partial(_body, b_per_w, n_chunks, n_remap),
        out_type=jax.ShapeDtypeStruct((SEQ, batch, DIM), jnp.float32),
        mesh=mesh,
        scratch_types=(
            [pltpu.VMEM((b_per_w,), jnp.int32),
             pltpu.VMEM((b_per_w,), jnp.int32),
             pltpu.VMEM((n_remap // NS,), jnp.int32),
             pltpu.VMEM_SHARED((n_remap,), jnp.int32)]
            + [pltpu.VMEM((K, DIM), jnp.float32)] * NBUF
            + [pltpu.SemaphoreType.DMA] * (3 * NBUF)
        ),
    )
    return fn(ids_perm, remap, emb)


def kernel(input_ids, attention_mask, embedding_dict, input_ids2dict_ids):
    batch, seq = input_ids.shape
    ids_perm = (input_ids.astype(jnp.int32)
                .reshape(NW, K, seq).transpose(0, 2, 1).reshape(-1))
    remap = input_ids2dict_ids.astype(jnp.int32)
    pad = (-remap.shape[0]) % (8 * NS)
    remap = jnp.pad(remap, (0, pad))
    out_t = _lookup(ids_perm, remap, embedding_dict)
    return (out_t.transpose(1, 0, 2), attention_mask)
